# baseline (speedup 1.0000x reference)
import math

import jax
import jax.numpy as jnp
from jax.experimental import pallas as pl

_NH = 4
_R = 8
_HEADS = 4
_CK = 32
_VH = 16

_bf16 = jnp.bfloat16
_f32 = jnp.float32


def _dot(a, b):
    return jnp.dot(a, b, preferred_element_type=_f32)


def _rnd(a):
    return a.astype(_bf16).astype(_f32)


def _fused_kernel(x_ref, hs_ref, remb_ref, pa_Wq_ref, pa_bq_ref, pa_Wk_ref,
                  pa_bk_ref, pa_Wv_ref, pa_bv_ref, sa_Wq_ref, sa_Wk_ref,
                  Wih_ref, Whh_ref, bih_ref, bhh_ref, cq_ref, ck_ref, cv_ref,
                  S3_ref, Gden_ref, Expand_ref, out_ref):
    Bt = x_ref.shape[0]
    RT = Bt * _NH
    Hd = hs_ref.shape[1]
    KD = pa_Wq_ref.shape[1]
    VD = pa_Wv_ref.shape[1]

    xb_bf = x_ref[...].astype(_bf16)
    hsb = hs_ref[...]
    hs_bf = hsb.astype(_bf16)

    q = _dot(hs_bf, pa_Wq_ref[...]) + pa_bq_ref[...]
    k0 = _dot(xb_bf, pa_Wk_ref[...]) + pa_bk_ref[...]
    v0 = _dot(xb_bf, pa_Wv_ref[...]) + pa_bv_ref[...]
    k0r = _rnd(k0)
    v0r = _rnd(v0)
    krep = jnp.broadcast_to(k0r[:, None, :], (Bt, _NH, KD)).reshape(RT, KD)
    vrep = jnp.broadcast_to(v0r[:, None, :], (Bt, _NH, VD)).reshape(RT, VD)
    pa_scale = 1.0 / math.sqrt(KD)
    qr = _rnd(q)
    l0 = jnp.sum(qr * krep, axis=1, keepdims=True) * pa_scale
    l1 = jnp.sum(qr * _rnd(pa_bk_ref[...]), axis=1, keepdims=True) * pa_scale
    lm = jnp.maximum(l0, l1)
    a0 = jnp.exp(l0 - lm)
    a1 = jnp.exp(l1 - lm)
    inv = 1.0 / (a0 + a1)
    attn0 = a0 * inv
    attn1 = a1 * inv
    inputs_b = _rnd(attn0) * vrep + _rnd(attn1) * _rnd(pa_bv_ref[...])
    inputs_bf = inputs_b.astype(_bf16)

    cat_ih = jnp.concatenate([hs_bf, inputs_bf], axis=1)
    qs = _dot(cat_ih, sa_Wq_ref[...])
    ksel = _dot(remb_ref[...].astype(_bf16), sa_Wk_ref[...])
    scores = jax.lax.dot_general(
        qs.astype(_bf16), ksel.astype(_bf16), (((1,), (1,)), ((), ())),
        preferred_element_type=_f32) * (1.0 / math.sqrt(qs.shape[1]))
    smax = jnp.max(scores, axis=1, keepdims=True)
    lane = jax.lax.broadcasted_iota(jnp.int32, scores.shape, 1)
    cand = jnp.where(scores == smax, lane, _R)
    sel = jnp.min(cand, axis=1, keepdims=True)
    mask_bf = (lane == sel).astype(_bf16)

    xbig = jnp.concatenate(
        [mask_bf[:, r:r + 1] * inputs_bf for r in range(_R)], axis=1)
    hbig = jnp.concatenate(
        [mask_bf[:, r:r + 1] * hs_bf for r in range(_R)], axis=1)
    px = _dot(xbig, Wih_ref[...]) + _dot(mask_bf, bih_ref[...])
    ph = _dot(hbig, Whh_ref[...]) + _dot(mask_bf, bhh_ref[...])
    rg = jax.nn.sigmoid(px[:, 0:Hd] + ph[:, 0:Hd])
    zg = jax.nn.sigmoid(px[:, Hd:2 * Hd] + ph[:, Hd:2 * Hd])
    ng = jnp.tanh(px[:, 2 * Hd:3 * Hd] + rg * ph[:, 2 * Hd:3 * Hd])
    hnew = (1.0 - zg) * ng + zg * hsb

    h3 = hnew.reshape(Bt, _NH, Hd)
    hcat = jnp.concatenate([h3[:, n, :] for n in range(_NH)], axis=1)
    hcat_bf = hcat.astype(_bf16)
    QK = _HEADS * _CK
    qcat = _dot(hcat_bf, cq_ref[...]).astype(_bf16)
    kcat = _dot(hcat_bf, ck_ref[...]).astype(_bf16)
    vcat = _rnd(_dot(hcat_bf, cv_ref[...]))
    qrep = jnp.concatenate(
        [qcat[:, n * QK:(n + 1) * QK] for n in range(_NH) for _ in range(_NH)],
        axis=1)
    krep2 = jnp.concatenate([kcat] * _NH, axis=1)
    ls = _dot(qrep * krep2, S3_ref[...]) * (1.0 / math.sqrt(_CK))
    e4 = jnp.exp(ls)
    den = _dot(e4.astype(_bf16), Gden_ref[...])
    p = (e4 * (1.0 / den)).astype(_bf16)
    pexp = _dot(p, Expand_ref[...])
    vtile = jnp.concatenate([vcat] * _NH, axis=1)
    prod2 = pexp * vtile
    ctx_parts = []
    for n in range(_NH):
        acc = prod2[:, n * _NH * Hd:n * _NH * Hd + Hd]
        for m in range(1, _NH):
            lo = n * _NH * Hd + m * Hd
            acc = acc + prod2[:, lo:lo + Hd]
        ctx_parts.append(acc)
    out_ref[...] = hcat + jnp.concatenate(ctx_parts, axis=1)


def kernel(x, hs, rule_embeddings, pa_Wq, pa_bq, pa_Wk, pa_bk, pa_Wv, pa_bv,
           sa_Wq, sa_Wk, gru_Wih, gru_Whh, gru_bih, gru_bhh, c_Wq, c_Wk, c_Wv):
    B, IN = x.shape
    NH, Hd = hs.shape[1], hs.shape[2]
    Bt = 2048
    G = B // Bt
    hs_f = hs.reshape(B * NH, Hd)
    bf = lambda a: a.astype(_bf16)
    Wih_all = bf(gru_Wih).transpose(0, 2, 1).reshape(_R * gru_Wih.shape[2], -1)
    Whh_all = bf(gru_Whh).transpose(0, 2, 1).reshape(_R * gru_Whh.shape[2], -1)
    eye = jnp.eye(NH, dtype=_bf16)
    cq_blk = jnp.kron(eye, bf(c_Wq))
    ck_blk = jnp.kron(eye, bf(c_Wk))
    cv_blk = jnp.kron(eye, bf(c_Wv))

    QK = _HEADS * _CK
    l_i = jnp.arange(NH * NH * QK)[:, None]
    j_c = jnp.arange(NH * _HEADS * NH)[None, :]
    S3 = bf((l_i // (NH * QK) == j_c // (_HEADS * NH))
            & ((l_i // QK) % NH == j_c % NH)
            & ((l_i % QK) // _CK == (j_c // NH) % _HEADS))
    j2 = jnp.arange(NH * _HEADS * NH)
    Gden = bf(j2[:, None] // NH == j2[None, :] // NH)
    j_r = jnp.arange(NH * _HEADS * NH)[:, None]
    e_c = jnp.arange(NH * NH * _HEADS * _VH)[None, :]
    Expand = bf((j_r // (_HEADS * NH) == e_c // (NH * Hd))
                & (j_r % NH == (e_c // (_HEADS * _VH)) % NH)
                & ((j_r // NH) % _HEADS == (e_c // _VH) % _HEADS))
    def row2(v):
        return v.reshape(1, -1)

    full2 = lambda a: pl.BlockSpec(a.shape, lambda i: (0, 0))
    args = (x, hs_f, rule_embeddings, bf(pa_Wq), row2(pa_bq), bf(pa_Wk),
            row2(pa_bk), bf(pa_Wv), row2(pa_bv), bf(sa_Wq), bf(sa_Wk),
            Wih_all, Whh_all, bf(gru_bih), bf(gru_bhh), cq_blk, ck_blk,
            cv_blk, S3, Gden, Expand)
    out = pl.pallas_call(
        _fused_kernel,
        grid=(G,),
        in_specs=[
            pl.BlockSpec((Bt, IN), lambda i: (i, 0)),
            pl.BlockSpec((Bt * NH, Hd), lambda i: (i, 0)),
        ] + [full2(a) for a in args[2:]],
        out_specs=pl.BlockSpec((Bt, NH * Hd), lambda i: (i, 0)),
        out_shape=jax.ShapeDtypeStruct((B, NH * Hd), jnp.float32),
    )(*args)
    return out.reshape(B, NH, Hd)


# speedup vs baseline: 1.3045x; 1.3045x over previous
import math

import jax
import jax.numpy as jnp
from jax.experimental import pallas as pl

_NH = 4
_R = 8
_HEADS = 4
_CK = 32
_VH = 16

_bf16 = jnp.bfloat16
_f32 = jnp.float32


def _dot(a, b):
    return jnp.dot(a, b, preferred_element_type=_f32)


def _rnd(a):
    return a.astype(_bf16).astype(_f32)


def _fused_kernel(x_ref, hs_ref, remb_ref, pa_Wq_ref, pa_bq_ref, pa_Wk_ref,
                  pa_bk_ref, pa_Wv_ref, pa_bv_ref, sa_Wq_ref, sa_Wk_ref,
                  Wih_ref, Whh_ref, bih_ref, bhh_ref, cq_ref, ck_ref, cv_ref,
                  S3_ref, Gden_ref, Expand_ref, out_ref):
    Bt = x_ref.shape[0]
    RT = Bt * _NH
    Hd = hs_ref.shape[1]
    KD = pa_Wq_ref.shape[1]
    VD = pa_Wv_ref.shape[1]

    xb_bf = x_ref[...].astype(_bf16)
    hsb = hs_ref[...]
    hs_bf = hsb.astype(_bf16)

    q = _dot(hs_bf, pa_Wq_ref[...]) + pa_bq_ref[...]
    k0 = _dot(xb_bf, pa_Wk_ref[...]) + pa_bk_ref[...]
    v0 = _dot(xb_bf, pa_Wv_ref[...]) + pa_bv_ref[...]
    k0r = _rnd(k0)
    v0r = _rnd(v0)
    krep = jnp.broadcast_to(k0r[:, None, :], (Bt, _NH, KD)).reshape(RT, KD)
    vrep = jnp.broadcast_to(v0r[:, None, :], (Bt, _NH, VD)).reshape(RT, VD)
    pa_scale = 1.0 / math.sqrt(KD)
    qr = _rnd(q)
    l0 = jnp.sum(qr * krep, axis=1, keepdims=True) * pa_scale
    l1 = jnp.sum(qr * _rnd(pa_bk_ref[...]), axis=1, keepdims=True) * pa_scale
    lm = jnp.maximum(l0, l1)
    a0 = jnp.exp(l0 - lm)
    a1 = jnp.exp(l1 - lm)
    inv = 1.0 / (a0 + a1)
    attn0 = a0 * inv
    attn1 = a1 * inv
    inputs_b = _rnd(attn0) * vrep + _rnd(attn1) * _rnd(pa_bv_ref[...])
    inputs_bf = inputs_b.astype(_bf16)

    cat_ih = jnp.concatenate([hs_bf, inputs_bf], axis=1)
    qs = _dot(cat_ih, sa_Wq_ref[...])
    ksel = _dot(remb_ref[...].astype(_bf16), sa_Wk_ref[...])
    scores = jax.lax.dot_general(
        qs.astype(_bf16), ksel.astype(_bf16), (((1,), (1,)), ((), ())),
        preferred_element_type=_f32) * (1.0 / math.sqrt(qs.shape[1]))
    smax = jnp.max(scores, axis=1, keepdims=True)
    lane = jax.lax.broadcasted_iota(jnp.int32, scores.shape, 1)
    cand = jnp.where(scores == smax, lane, _R)
    sel = jnp.min(cand, axis=1, keepdims=True)
    mask_bf = (lane == sel).astype(_bf16)

    xbig = jnp.concatenate(
        [mask_bf[:, r:r + 1] * inputs_bf for r in range(_R)], axis=1)
    hbig = jnp.concatenate(
        [mask_bf[:, r:r + 1] * hs_bf for r in range(_R)], axis=1)
    px = _dot(xbig, Wih_ref[...]) + _dot(mask_bf, bih_ref[...])
    ph = _dot(hbig, Whh_ref[...]) + _dot(mask_bf, bhh_ref[...])
    rg = jax.nn.sigmoid(px[:, 0:Hd] + ph[:, 0:Hd])
    zg = jax.nn.sigmoid(px[:, Hd:2 * Hd] + ph[:, Hd:2 * Hd])
    ng = jnp.tanh(px[:, 2 * Hd:3 * Hd] + rg * ph[:, 2 * Hd:3 * Hd])
    hnew = (1.0 - zg) * ng + zg * hsb

    h3 = hnew.reshape(Bt, _NH, Hd)
    hcat = jnp.concatenate([h3[:, n, :] for n in range(_NH)], axis=1)
    hcat_bf = hcat.astype(_bf16)
    QK = _HEADS * _CK
    qcat = _dot(hcat_bf, cq_ref[...]).astype(_bf16)
    kcat = _dot(hcat_bf, ck_ref[...]).astype(_bf16)
    vcat = _rnd(_dot(hcat_bf, cv_ref[...]))
    qrep = jnp.concatenate(
        [qcat[:, n * QK:(n + 1) * QK] for n in range(_NH) for _ in range(_NH)],
        axis=1)
    krep2 = jnp.concatenate([kcat] * _NH, axis=1)
    ls = _dot(qrep * krep2, S3_ref[...]) * (1.0 / math.sqrt(_CK))
    e4 = jnp.exp(ls)
    den = _dot(e4.astype(_bf16), Gden_ref[...])
    p = (e4 * (1.0 / den)).astype(_bf16)
    pexp = _dot(p, Expand_ref[...])
    vtile = jnp.concatenate([vcat] * _NH, axis=1)
    prod2 = pexp * vtile
    ctx_parts = []
    for n in range(_NH):
        acc = prod2[:, n * _NH * Hd:n * _NH * Hd + Hd]
        for m in range(1, _NH):
            lo = n * _NH * Hd + m * Hd
            acc = acc + prod2[:, lo:lo + Hd]
        ctx_parts.append(acc)
    out_ref[...] = hcat + jnp.concatenate(ctx_parts, axis=1)


def kernel(x, hs, rule_embeddings, pa_Wq, pa_bq, pa_Wk, pa_bk, pa_Wv, pa_bv,
           sa_Wq, sa_Wk, gru_Wih, gru_Whh, gru_bih, gru_bhh, c_Wq, c_Wk, c_Wv):
    B, IN = x.shape
    NH, Hd = hs.shape[1], hs.shape[2]
    Bt = 1024
    G = B // Bt
    hs_f = hs.reshape(B * NH, Hd)
    bf = lambda a: a.astype(_bf16)
    Wih_all = bf(gru_Wih).transpose(0, 2, 1).reshape(_R * gru_Wih.shape[2], -1)
    Whh_all = bf(gru_Whh).transpose(0, 2, 1).reshape(_R * gru_Whh.shape[2], -1)
    eye = jnp.eye(NH, dtype=_bf16)
    cq_blk = jnp.kron(eye, bf(c_Wq))
    ck_blk = jnp.kron(eye, bf(c_Wk))
    cv_blk = jnp.kron(eye, bf(c_Wv))

    QK = _HEADS * _CK
    l_i = jnp.arange(NH * NH * QK)[:, None]
    j_c = jnp.arange(NH * _HEADS * NH)[None, :]
    S3 = bf((l_i // (NH * QK) == j_c // (_HEADS * NH))
            & ((l_i // QK) % NH == j_c % NH)
            & ((l_i % QK) // _CK == (j_c // NH) % _HEADS))
    j2 = jnp.arange(NH * _HEADS * NH)
    Gden = bf(j2[:, None] // NH == j2[None, :] // NH)
    j_r = jnp.arange(NH * _HEADS * NH)[:, None]
    e_c = jnp.arange(NH * NH * _HEADS * _VH)[None, :]
    Expand = bf((j_r // (_HEADS * NH) == e_c // (NH * Hd))
                & (j_r % NH == (e_c // (_HEADS * _VH)) % NH)
                & ((j_r // NH) % _HEADS == (e_c // _VH) % _HEADS))
    def row2(v):
        return v.reshape(1, -1)

    full2 = lambda a: pl.BlockSpec(a.shape, lambda i: (0, 0))
    args = (x, hs_f, rule_embeddings, bf(pa_Wq), row2(pa_bq), bf(pa_Wk),
            row2(pa_bk), bf(pa_Wv), row2(pa_bv), bf(sa_Wq), bf(sa_Wk),
            Wih_all, Whh_all, bf(gru_bih), bf(gru_bhh), cq_blk, ck_blk,
            cv_blk, S3, Gden, Expand)
    out = pl.pallas_call(
        _fused_kernel,
        grid=(G,),
        in_specs=[
            pl.BlockSpec((Bt, IN), lambda i: (i, 0)),
            pl.BlockSpec((Bt * NH, Hd), lambda i: (i, 0)),
        ] + [full2(a) for a in args[2:]],
        out_specs=pl.BlockSpec((Bt, NH * Hd), lambda i: (i, 0)),
        out_shape=jax.ShapeDtypeStruct((B, NH * Hd), jnp.float32),
    )(*args)
    return out.reshape(B, NH, Hd)


# speedup vs baseline: 1.4197x; 1.0883x over previous
import math

import jax
import jax.numpy as jnp
from jax.experimental import pallas as pl

_NH = 4
_R = 8
_HEADS = 4
_CK = 32
_VH = 16

_bf16 = jnp.bfloat16
_f32 = jnp.float32


def _dot(a, b):
    return jnp.dot(a, b, preferred_element_type=_f32)


def _rnd(a):
    return a.astype(_bf16).astype(_f32)


def _fused_kernel(x_ref, hs_ref, remb_ref, pa_Wq_ref, pa_bq_ref, pa_Wk_ref,
                  pa_bk_ref, pa_Wv_ref, pa_bv_ref, sa_Wq_ref, sa_Wk_ref,
                  Wih_ref, Whh_ref, bih_ref, bhh_ref, cq_ref, ck_ref, cv_ref,
                  S3_ref, Gden_ref, Expand_ref, out_ref):
    Bt = x_ref.shape[0]
    RT = Bt * _NH
    Hd = hs_ref.shape[1]
    KD = pa_Wq_ref.shape[1]
    VD = pa_Wv_ref.shape[1]

    xb_bf = x_ref[...].astype(_bf16)
    hsb = hs_ref[...]
    hs_bf = hsb.astype(_bf16)

    q = _dot(hs_bf, pa_Wq_ref[...]) + pa_bq_ref[...]
    k0 = _dot(xb_bf, pa_Wk_ref[...]) + pa_bk_ref[...]
    v0 = _dot(xb_bf, pa_Wv_ref[...]) + pa_bv_ref[...]
    k0r = _rnd(k0)
    v0r = _rnd(v0)
    krep = jnp.broadcast_to(k0r[:, None, :], (Bt, _NH, KD)).reshape(RT, KD)
    vrep = jnp.broadcast_to(v0r[:, None, :], (Bt, _NH, VD)).reshape(RT, VD)
    pa_scale = 1.0 / math.sqrt(KD)
    qr = _rnd(q)
    l0 = jnp.sum(qr * krep, axis=1, keepdims=True) * pa_scale
    l1 = jnp.sum(qr * _rnd(pa_bk_ref[...]), axis=1, keepdims=True) * pa_scale
    attn0 = jax.nn.sigmoid(l0 - l1)
    attn1 = 1.0 - attn0
    inputs_b = _rnd(attn0) * vrep + _rnd(attn1) * _rnd(pa_bv_ref[...])
    inputs_bf = inputs_b.astype(_bf16)

    cat_ih = jnp.concatenate([hs_bf, inputs_bf], axis=1)
    qs = _dot(cat_ih, sa_Wq_ref[...])
    ksel = _dot(remb_ref[...].astype(_bf16), sa_Wk_ref[...])
    scores = jax.lax.dot_general(
        qs.astype(_bf16), ksel.astype(_bf16), (((1,), (1,)), ((), ())),
        preferred_element_type=_f32) * (1.0 / math.sqrt(qs.shape[1]))
    lane = jax.lax.broadcasted_iota(jnp.int32, scores.shape, 1)
    sel = jnp.argmax(scores, axis=1)[:, None]
    mask_bf = (lane == sel).astype(_bf16)

    xbig = jnp.concatenate(
        [mask_bf[:, r:r + 1] * inputs_bf for r in range(_R)], axis=1)
    hbig = jnp.concatenate(
        [mask_bf[:, r:r + 1] * hs_bf for r in range(_R)], axis=1)
    px = _dot(xbig, Wih_ref[...]) + _dot(mask_bf, bih_ref[...])
    ph = _dot(hbig, Whh_ref[...]) + _dot(mask_bf, bhh_ref[...])
    rg = jax.nn.sigmoid(px[:, 0:Hd] + ph[:, 0:Hd])
    zg = jax.nn.sigmoid(px[:, Hd:2 * Hd] + ph[:, Hd:2 * Hd])
    ng = jnp.tanh(px[:, 2 * Hd:3 * Hd] + rg * ph[:, 2 * Hd:3 * Hd])
    hnew = (1.0 - zg) * ng + zg * hsb

    h3 = hnew.reshape(Bt, _NH, Hd)
    hcat = jnp.concatenate([h3[:, n, :] for n in range(_NH)], axis=1)
    hcat_bf = hcat.astype(_bf16)
    QK = _HEADS * _CK
    qcat = _dot(hcat_bf, cq_ref[...]).astype(_bf16)
    kcat = _dot(hcat_bf, ck_ref[...]).astype(_bf16)
    vcat = _rnd(_dot(hcat_bf, cv_ref[...]))
    qrep = jnp.concatenate(
        [qcat[:, n * QK:(n + 1) * QK] for n in range(_NH) for _ in range(_NH)],
        axis=1)
    krep2 = jnp.concatenate([kcat] * _NH, axis=1)
    ls = _dot(qrep * krep2, S3_ref[...]) * (1.0 / math.sqrt(_CK))
    e4 = jnp.exp(ls)
    den = _dot(e4.astype(_bf16), Gden_ref[...])
    p = (e4 * (1.0 / den)).astype(_bf16)
    pexp = _dot(p, Expand_ref[...])
    vtile = jnp.concatenate([vcat] * _NH, axis=1)
    prod2 = pexp * vtile
    ctx_parts = []
    for n in range(_NH):
        acc = prod2[:, n * _NH * Hd:n * _NH * Hd + Hd]
        for m in range(1, _NH):
            lo = n * _NH * Hd + m * Hd
            acc = acc + prod2[:, lo:lo + Hd]
        ctx_parts.append(acc)
    out_ref[...] = hcat + jnp.concatenate(ctx_parts, axis=1)


def kernel(x, hs, rule_embeddings, pa_Wq, pa_bq, pa_Wk, pa_bk, pa_Wv, pa_bv,
           sa_Wq, sa_Wk, gru_Wih, gru_Whh, gru_bih, gru_bhh, c_Wq, c_Wk, c_Wv):
    B, IN = x.shape
    NH, Hd = hs.shape[1], hs.shape[2]
    Bt = 1024
    G = B // Bt
    hs_f = hs.reshape(B * NH, Hd)
    bf = lambda a: a.astype(_bf16)
    Wih_all = bf(gru_Wih).transpose(0, 2, 1).reshape(_R * gru_Wih.shape[2], -1)
    Whh_all = bf(gru_Whh).transpose(0, 2, 1).reshape(_R * gru_Whh.shape[2], -1)
    eye = jnp.eye(NH, dtype=_bf16)
    cq_blk = jnp.kron(eye, bf(c_Wq))
    ck_blk = jnp.kron(eye, bf(c_Wk))
    cv_blk = jnp.kron(eye, bf(c_Wv))

    QK = _HEADS * _CK
    l_i = jnp.arange(NH * NH * QK)[:, None]
    j_c = jnp.arange(NH * _HEADS * NH)[None, :]
    S3 = bf((l_i // (NH * QK) == j_c // (_HEADS * NH))
            & ((l_i // QK) % NH == j_c % NH)
            & ((l_i % QK) // _CK == (j_c // NH) % _HEADS))
    j2 = jnp.arange(NH * _HEADS * NH)
    Gden = bf(j2[:, None] // NH == j2[None, :] // NH)
    j_r = jnp.arange(NH * _HEADS * NH)[:, None]
    e_c = jnp.arange(NH * NH * _HEADS * _VH)[None, :]
    Expand = bf((j_r // (_HEADS * NH) == e_c // (NH * Hd))
                & (j_r % NH == (e_c // (_HEADS * _VH)) % NH)
                & ((j_r // NH) % _HEADS == (e_c // _VH) % _HEADS))
    def row2(v):
        return v.reshape(1, -1)

    full2 = lambda a: pl.BlockSpec(a.shape, lambda i: (0, 0))
    args = (x, hs_f, rule_embeddings, bf(pa_Wq), row2(pa_bq), bf(pa_Wk),
            row2(pa_bk), bf(pa_Wv), row2(pa_bv), bf(sa_Wq), bf(sa_Wk),
            Wih_all, Whh_all, bf(gru_bih), bf(gru_bhh), cq_blk, ck_blk,
            cv_blk, S3, Gden, Expand)
    out = pl.pallas_call(
        _fused_kernel,
        grid=(G,),
        in_specs=[
            pl.BlockSpec((Bt, IN), lambda i: (i, 0)),
            pl.BlockSpec((Bt * NH, Hd), lambda i: (i, 0)),
        ] + [full2(a) for a in args[2:]],
        out_specs=pl.BlockSpec((Bt, NH * Hd), lambda i: (i, 0)),
        out_shape=jax.ShapeDtypeStruct((B, NH * Hd), jnp.float32),
    )(*args)
    return out.reshape(B, NH, Hd)
